# baseline (speedup 1.0000x reference)
import jax
import jax.numpy as jnp
import numpy as np
from jax.experimental import pallas as pl
from jax.experimental.pallas import tpu as pltpu
from jax.sharding import Mesh, PartitionSpec as P

_LAMBDA = 1e-4
_CBLK = 128


def _shift_w(v, d):
    n = v.shape[1]
    zshape = (v.shape[0], abs(d), v.shape[2])
    z = jnp.zeros(zshape, v.dtype)
    if d > 0:
        return jnp.concatenate([z, v[:, : n - d, :]], axis=1)
    return jnp.concatenate([v[:, -d:, :], z], axis=1)


def _box_w3(v):
    vl = _shift_w(v, 1)
    vr = _shift_w(v, -1)
    s1l = v + vl
    s1r = v + vr
    b3 = s1l + vr
    b7 = b3 + _shift_w(s1l, 2) + _shift_w(s1r, -2)
    b11 = b7 + _shift_w(s1l, 4) + _shift_w(s1r, -4)
    return b3, b7, b11


def _box_h(v, p):
    nh = v.shape[0]
    acc = v[0:1]
    cums = [acc]
    for h in range(1, nh):
        acc = acc + v[h : h + 1]
        cums.append(acc)
    out = []
    for h in range(nh):
        hi = cums[min(h + p, nh - 1)]
        lo_idx = h - p - 1
        out.append(hi - cums[lo_idx] if lo_idx >= 0 else hi)
    return jnp.concatenate(out, axis=0)


def _camsam_body(x_ref, o_ref):
    x = x_ref[...]
    x2 = x * x
    xw = _box_w3(x)
    sw = _box_w3(x2)
    energy = None
    for i, k in enumerate((3, 7, 11)):
        p = k // 2
        inv = 1.0 / float(k * k)
        m = _box_h(xw[i], p) * inv
        m2 = _box_h(sw[i], p) * inv
        var = m2 - m * m
        num = (x - m) * (x - m)
        e = num / (4.0 * (var + _LAMBDA))
        energy = e if energy is None else energy + e
    att = jax.nn.sigmoid(0.5 - energy * (1.0 / 3.0))
    o_ref[...] = x * att


def _forward(x):
    n, c, h, w = x.shape
    xt = jnp.transpose(x, (0, 2, 3, 1)).reshape(n * h, w, c)
    csplit = c // _CBLK
    grid = (n * csplit,)
    out = pl.pallas_call(
        _camsam_body,
        grid=grid,
        in_specs=[
            pl.BlockSpec((h, w, _CBLK), lambda i, s=csplit: (i // s, 0, i % s))
        ],
        out_specs=pl.BlockSpec(
            (h, w, _CBLK), lambda i, s=csplit: (i // s, 0, i % s)
        ),
        out_shape=jax.ShapeDtypeStruct((n * h, w, c), x.dtype),
        compiler_params=pltpu.CompilerParams(
            dimension_semantics=("parallel",),
            vmem_limit_bytes=100 * 1024 * 1024,
        ),
    )(xt)
    return jnp.transpose(out.reshape(n, h, w, c), (0, 3, 1, 2))


def kernel(x):
    devs = jax.devices()
    n = x.shape[0]
    if len(devs) < 2 or n % 2 != 0:
        return _forward(x)
    mesh = Mesh(np.array(devs[:2]), ("d",))
    xs = jax.lax.with_sharding_constraint(
        x, jax.sharding.NamedSharding(mesh, P("d"))
    )
    return jax.shard_map(
        _forward, mesh=mesh, in_specs=P("d"), out_specs=P("d"),
        check_vma=False,
    )(xs)


# speedup vs baseline: 1.8786x; 1.8786x over previous
import jax
import jax.numpy as jnp
import numpy as np
from jax.experimental import pallas as pl
from jax.experimental.pallas import tpu as pltpu
from jax.sharding import Mesh, PartitionSpec as P

_LAMBDA = 1e-4
_CBLK = 128


def _shift_w(v, d):
    n = v.shape[1]
    zshape = (v.shape[0], abs(d), v.shape[2])
    z = jnp.zeros(zshape, v.dtype)
    if d > 0:
        return jnp.concatenate([z, v[:, : n - d, :]], axis=1)
    return jnp.concatenate([v[:, -d:, :], z], axis=1)


def _box_w3(v):
    vl = _shift_w(v, 1)
    vr = _shift_w(v, -1)
    s1l = v + vl
    s1r = v + vr
    b3 = s1l + vr
    b7 = b3 + _shift_w(s1l, 2) + _shift_w(s1r, -2)
    b11 = b7 + _shift_w(s1l, 4) + _shift_w(s1r, -4)
    return b3, b7, b11


def _box_h(v, p):
    nh = v.shape[0]
    acc = v[0:1]
    cums = [acc]
    for h in range(1, nh):
        acc = acc + v[h : h + 1]
        cums.append(acc)
    out = []
    for h in range(nh):
        hi = cums[min(h + p, nh - 1)]
        lo_idx = h - p - 1
        out.append(hi - cums[lo_idx] if lo_idx >= 0 else hi)
    return jnp.concatenate(out, axis=0)


def _camsam_body(x_ref, o_ref):
    x = x_ref[...]
    x2 = x * x
    xw = _box_w3(x)
    sw = _box_w3(x2)
    energy = None
    for i, k in enumerate((3, 7, 11)):
        p = k // 2
        inv = 1.0 / float(k * k)
        m = _box_h(xw[i], p) * inv
        m2 = _box_h(sw[i], p) * inv
        var = m2 - m * m
        num = (x - m) * (x - m)
        e = num / (var + _LAMBDA)
        energy = e if energy is None else energy + e
    att = jax.nn.sigmoid(0.5 - energy * (1.0 / 12.0))
    o_ref[...] = x * att


def _forward(x):
    n, c, h, w = x.shape
    xt = jnp.transpose(x, (0, 2, 3, 1)).reshape(n * h, w, c)
    csplit = c // _CBLK
    grid = (n * csplit,)
    out = pl.pallas_call(
        _camsam_body,
        grid=grid,
        in_specs=[
            pl.BlockSpec((h, w, _CBLK), lambda i, s=csplit: (i // s, 0, i % s))
        ],
        out_specs=pl.BlockSpec(
            (h, w, _CBLK), lambda i, s=csplit: (i // s, 0, i % s)
        ),
        out_shape=jax.ShapeDtypeStruct((n * h, w, c), x.dtype),
        compiler_params=pltpu.CompilerParams(
            dimension_semantics=("parallel",),
            vmem_limit_bytes=100 * 1024 * 1024,
            allow_input_fusion=[True],
        ),
    )(xt)
    return jnp.transpose(out.reshape(n, h, w, c), (0, 3, 1, 2))


def kernel(x):
    return _forward(x)
